# Initial kernel scaffold; baseline (speedup 1.0000x reference)
import jax
import jax.numpy as jnp
from jax.experimental import pallas as pl


def kernel(x, edge_index, W, b):
    raise NotImplementedError("write your pallas kernel here")


# speedup vs baseline: 13.2606x; 13.2606x over previous
import functools

import jax
import jax.numpy as jnp
from jax import lax
from jax.experimental import pallas as pl
from jax.experimental.pallas import tpu as pltpu
from jax.experimental.pallas import tpu_sc as plsc

_N = 10000
_E = 320000
_D = 128
_C = 10

_NPAD = 10240
_WIN = 128
_NWIN = _E // _WIN
_NTILES = 32
_WBASE = _NWIN // _NTILES
_WEXTRA = _NWIN - _WBASE * _NTILES
_CHUNK = _NPAD // 16

_mesh = plsc.VectorSubcoreMesh(core_axis_name="c", subcore_axis_name="s")


def _zero_vmem(buf, n):
    z = jnp.zeros((16,), jnp.float32)
    for i in range(n // 16):
        buf[pl.ds(i * 16, 16)] = z


@functools.partial(
    pl.kernel,
    mesh=_mesh,
    out_type=jax.ShapeDtypeStruct((2, _NPAD), jnp.float32),
    scratch_types=[
        pltpu.VMEM((_CHUNK,), jnp.float32),
        pltpu.VMEM((_WIN,), jnp.float32),
        pltpu.VMEM((_WIN,), jnp.int32),
        pltpu.VMEM_SHARED((_NPAD,), jnp.float32),
    ],
)
def _deg_kernel(ei, deg_out, zbuf, ones, idx, deg_sp):
    cid = lax.axis_index("c")
    sid = lax.axis_index("s")
    gid = cid * 16 + sid

    _zero_vmem(zbuf, _CHUNK)
    pltpu.sync_copy(zbuf, deg_sp.at[pl.ds(sid * _CHUNK, _CHUNK)])
    o = jnp.ones((16,), jnp.float32)
    for i in range(_WIN // 16):
        ones[pl.ds(i * 16, 16)] = o
    plsc.subcore_barrier()

    base_w = gid * _WBASE + jnp.minimum(gid, _WEXTRA)

    def window(w):
        off = (base_w + w) * _WIN
        pltpu.sync_copy(ei.at[1, pl.ds(off, _WIN)], idx)
        pltpu.sync_copy(ones, deg_sp.at[idx], add=True)

    def body(w, carry):
        window(w)
        return carry

    lax.fori_loop(0, _WBASE, body, 0)

    @pl.when(gid < _WEXTRA)
    def _():
        window(_WBASE)

    plsc.subcore_barrier()
    pltpu.sync_copy(deg_sp.at[pl.ds(sid * _CHUNK, _CHUNK)], zbuf)
    pltpu.sync_copy(zbuf, deg_out.at[cid, pl.ds(sid * _CHUNK, _CHUNK)])


@functools.partial(
    pl.kernel,
    mesh=_mesh,
    out_type=jax.ShapeDtypeStruct((2, _NPAD), jnp.float32),
    scratch_types=[
        pltpu.VMEM((_CHUNK,), jnp.float32),
        pltpu.VMEM((_CHUNK,), jnp.float32),
        pltpu.VMEM((_WIN,), jnp.float32),
        pltpu.VMEM((_WIN,), jnp.int32),
        pltpu.VMEM((_WIN,), jnp.int32),
        pltpu.VMEM_SHARED((_NPAD,), jnp.float32),
        pltpu.VMEM_SHARED((_NPAD,), jnp.float32),
    ],
)
def _coef_kernel(ei, deg_part, coef_out,
                 d0, d1, wbuf, idxd, idxs, invd_sp, coef_sp):
    cid = lax.axis_index("c")
    sid = lax.axis_index("s")
    gid = cid * 16 + sid

    _zero_vmem(d1, _CHUNK)
    pltpu.sync_copy(d1, coef_sp.at[pl.ds(sid * _CHUNK, _CHUNK)])

    pltpu.sync_copy(deg_part.at[0, pl.ds(sid * _CHUNK, _CHUNK)], d0)
    pltpu.sync_copy(deg_part.at[1, pl.ds(sid * _CHUNK, _CHUNK)], d1)
    one = jnp.ones((16,), jnp.float32)
    for i in range(_CHUNK // 16):
        s = pl.ds(i * 16, 16)
        d0[s] = one / jnp.maximum(d0[s] + d1[s], one)
    pltpu.sync_copy(d0, invd_sp.at[pl.ds(sid * _CHUNK, _CHUNK)])
    plsc.subcore_barrier()

    base_w = gid * _WBASE + jnp.minimum(gid, _WEXTRA)

    def window(w):
        off = (base_w + w) * _WIN
        pltpu.sync_copy(ei.at[1, pl.ds(off, _WIN)], idxd)
        pltpu.sync_copy(ei.at[0, pl.ds(off, _WIN)], idxs)
        pltpu.sync_copy(invd_sp.at[idxd], wbuf)
        pltpu.sync_copy(wbuf, coef_sp.at[idxs], add=True)

    def body(w, carry):
        window(w)
        return carry

    lax.fori_loop(0, _WBASE, body, 0)

    @pl.when(gid < _WEXTRA)
    def _():
        window(_WBASE)

    plsc.subcore_barrier()
    pltpu.sync_copy(coef_sp.at[pl.ds(sid * _CHUNK, _CHUNK)], d0)
    pltpu.sync_copy(d0, coef_out.at[cid, pl.ds(sid * _CHUNK, _CHUNK)])


def _tc_body(cp_ref, x_ref, w_ref, b_ref, o_ref):
    coef = cp_ref[0:1, :_N] + cp_ref[1:2, :_N]
    v = jnp.dot(coef, x_ref[...], preferred_element_type=jnp.float32)
    csum = jnp.sum(coef)
    o_ref[...] = (jnp.dot(v, w_ref[...], preferred_element_type=jnp.float32)
                  + csum * b_ref[...]) * (1.0 / _N)


def kernel(x, edge_index, W, b):
    deg_part = _deg_kernel(edge_index)
    coef_part = _coef_kernel(edge_index, deg_part)
    out = pl.pallas_call(
        _tc_body,
        out_shape=jax.ShapeDtypeStruct((1, _C), jnp.float32),
    )(coef_part, x, W, b.reshape(1, _C))
    return out


# speedup vs baseline: 47.0037x; 3.5446x over previous
import functools

import jax
import jax.numpy as jnp
from jax import lax
from jax.experimental import pallas as pl
from jax.experimental.pallas import tpu as pltpu
from jax.experimental.pallas import tpu_sc as plsc

_N = 10000
_E = 320000
_D = 128
_C = 10

_NPAD = 10240
_WIN = 128
_NTILES = 32
_WCHUNK = 80
_NWIN = _NTILES * _WCHUNK
_EPAD = _NWIN * _WIN
_CHUNK = _NPAD // 16

_mesh = plsc.VectorSubcoreMesh(core_axis_name="c", subcore_axis_name="s")


def _zero_vmem(buf, n):
    z = jnp.zeros((16,), jnp.float32)
    for i in range(n // 16):
        buf[pl.ds(i * 16, 16)] = z


def _stage_indices(ei3, row, base_w, idx2):
    pltpu.sync_copy(ei3.at[row, pl.ds(base_w, _WCHUNK)], idx2)


@functools.partial(
    pl.kernel,
    mesh=_mesh,
    out_type=jax.ShapeDtypeStruct((2, _NPAD), jnp.float32),
    scratch_types=[
        pltpu.VMEM((_CHUNK,), jnp.float32),
        pltpu.VMEM((_WIN,), jnp.float32),
        pltpu.VMEM((_WCHUNK, _WIN), jnp.int32),
        pltpu.VMEM_SHARED((_NPAD,), jnp.float32),
        pltpu.SemaphoreType.DMA,
    ],
)
def _deg_kernel(ei3, deg_out, zbuf, ones, idx3, deg_sp, sem):
    cid = lax.axis_index("c")
    sid = lax.axis_index("s")
    gid = cid * 16 + sid
    base_w = pl.multiple_of(gid * _WCHUNK, 8)

    _zero_vmem(zbuf, _CHUNK)
    pltpu.sync_copy(zbuf, deg_sp.at[pl.ds(sid * _CHUNK, _CHUNK)])
    for i in range(_WIN // 16):
        ones[pl.ds(i * 16, 16)] = jnp.ones((16,), jnp.float32)
    _stage_indices(ei3, 1, base_w, idx3)
    plsc.subcore_barrier()

    def fire(j, c):
        pltpu.async_copy(ones, deg_sp.at[idx3.at[j]], sem, add=True)
        return c

    lax.fori_loop(0, _WCHUNK, fire, 0)

    def drain(j, c):
        pltpu.make_async_copy(ones, deg_sp.at[idx3.at[j]], sem).wait()
        return c

    lax.fori_loop(0, _WCHUNK, drain, 0)

    plsc.subcore_barrier()
    pltpu.sync_copy(deg_sp.at[pl.ds(sid * _CHUNK, _CHUNK)], zbuf)
    pltpu.sync_copy(zbuf, deg_out.at[cid, pl.ds(sid * _CHUNK, _CHUNK)])


@functools.partial(
    pl.kernel,
    mesh=_mesh,
    out_type=jax.ShapeDtypeStruct((2, _NPAD), jnp.float32),
    scratch_types=[
        pltpu.VMEM((_CHUNK,), jnp.float32),
        pltpu.VMEM((_CHUNK,), jnp.float32),
        pltpu.VMEM((_WCHUNK, _WIN), jnp.float32),
        pltpu.VMEM((_WCHUNK, _WIN), jnp.int32),
        pltpu.VMEM((_WCHUNK, _WIN), jnp.int32),
        pltpu.VMEM_SHARED((_NPAD,), jnp.float32),
        pltpu.VMEM_SHARED((_NPAD,), jnp.float32),
        pltpu.SemaphoreType.DMA,
        pltpu.SemaphoreType.DMA,
    ],
)
def _coef_kernel(ei3, deg_part, coef_out,
                 d0, d1, wtile, idxd3, idxs3, invd_sp, coef_sp, semg, sems):
    cid = lax.axis_index("c")
    sid = lax.axis_index("s")
    gid = cid * 16 + sid
    base_w = pl.multiple_of(gid * _WCHUNK, 8)

    _zero_vmem(d1, _CHUNK)
    pltpu.sync_copy(d1, coef_sp.at[pl.ds(sid * _CHUNK, _CHUNK)])

    pltpu.sync_copy(deg_part.at[0, pl.ds(sid * _CHUNK, _CHUNK)], d0)
    pltpu.sync_copy(deg_part.at[1, pl.ds(sid * _CHUNK, _CHUNK)], d1)
    one = jnp.ones((16,), jnp.float32)
    for i in range(_CHUNK // 16):
        s = pl.ds(i * 16, 16)
        d0[s] = one / jnp.maximum(d0[s] + d1[s], one)
    pltpu.sync_copy(d0, invd_sp.at[pl.ds(sid * _CHUNK, _CHUNK)])

    _stage_indices(ei3, 1, base_w, idxd3)
    _stage_indices(ei3, 0, base_w, idxs3)
    plsc.subcore_barrier()

    def fire_g(j, c):
        pltpu.async_copy(invd_sp.at[idxd3.at[j]], wtile.at[j], semg)
        return c

    lax.fori_loop(0, _WCHUNK, fire_g, 0)

    def drain_g(j, c):
        pltpu.make_async_copy(invd_sp.at[idxd3.at[j]], wtile.at[j],
                              semg).wait()
        return c

    lax.fori_loop(0, _WCHUNK, drain_g, 0)

    def fire_s(j, c):
        pltpu.async_copy(wtile.at[j], coef_sp.at[idxs3.at[j]], sems,
                         add=True)
        return c

    lax.fori_loop(0, _WCHUNK, fire_s, 0)

    def drain_s(j, c):
        pltpu.make_async_copy(wtile.at[j], coef_sp.at[idxs3.at[j]],
                              sems).wait()
        return c

    lax.fori_loop(0, _WCHUNK, drain_s, 0)

    plsc.subcore_barrier()
    pltpu.sync_copy(coef_sp.at[pl.ds(sid * _CHUNK, _CHUNK)], d0)
    pltpu.sync_copy(d0, coef_out.at[cid, pl.ds(sid * _CHUNK, _CHUNK)])


def _tc_body(cp_ref, x_ref, w_ref, b_ref, o_ref):
    coef = cp_ref[0:1, :_N] + cp_ref[1:2, :_N]
    v = jnp.dot(coef, x_ref[...], preferred_element_type=jnp.float32)
    csum = jnp.sum(coef)
    o_ref[...] = (jnp.dot(v, w_ref[...], preferred_element_type=jnp.float32)
                  + csum * b_ref[...]) * (1.0 / _N)


def kernel(x, edge_index, W, b):
    pad = (jnp.arange(_EPAD - _E, dtype=jnp.int32) % (_NPAD - _N)) + _N
    ei3 = jnp.concatenate([edge_index, jnp.stack([pad, pad])],
                          axis=1).reshape(2, _NWIN, _WIN)
    deg_part = _deg_kernel(ei3)
    coef_part = _coef_kernel(ei3, deg_part)
    out = pl.pallas_call(
        _tc_body,
        out_shape=jax.ShapeDtypeStruct((1, _C), jnp.float32),
    )(coef_part, x, W, b.reshape(1, _C))
    return out


# speedup vs baseline: 53.0361x; 1.1283x over previous
import functools

import jax
import jax.numpy as jnp
from jax import lax
from jax.experimental import pallas as pl
from jax.experimental.pallas import tpu as pltpu
from jax.experimental.pallas import tpu_sc as plsc

_N = 10000
_E = 320000
_D = 128
_C = 10

_NPAD = 10240
_WIN = 128
_NWIN = _E // _WIN
_NTILES = 32
_WBASE = _NWIN // _NTILES
_WEXTRA = _NWIN - _WBASE * _NTILES
_EBASE = _WBASE * _WIN
_CHUNK = _NPAD // 16

_mesh = plsc.VectorSubcoreMesh(core_axis_name="c", subcore_axis_name="s")


def _zero_vmem(buf, n):
    z = jnp.zeros((16,), jnp.float32)
    for i in range(n // 16):
        buf[pl.ds(i * 16, 16)] = z


def _stage(eif, row_base, base_e, gid, dst_f, sem):
    pltpu.async_copy(eif.at[pl.ds(row_base + base_e, _EBASE)],
                     dst_f.at[pl.ds(0, _EBASE)], sem)

    @pl.when(gid < _WEXTRA)
    def _():
        pltpu.async_copy(eif.at[pl.ds(row_base + base_e + _EBASE, _WIN)],
                         dst_f.at[pl.ds(_EBASE, _WIN)], sem)


def _stage_wait(eif, row_base, base_e, gid, dst_f, sem):
    pltpu.make_async_copy(eif.at[pl.ds(row_base + base_e, _EBASE)],
                          dst_f.at[pl.ds(0, _EBASE)], sem).wait()

    @pl.when(gid < _WEXTRA)
    def _():
        pltpu.make_async_copy(
            eif.at[pl.ds(row_base + base_e + _EBASE, _WIN)],
            dst_f.at[pl.ds(_EBASE, _WIN)], sem).wait()


@functools.partial(
    pl.kernel,
    mesh=_mesh,
    out_type=jax.ShapeDtypeStruct((2, _NPAD), jnp.float32),
    scratch_types=[
        pltpu.VMEM((_CHUNK,), jnp.float32),
        pltpu.VMEM((_WIN,), jnp.float32),
        pltpu.VMEM((_EBASE + _WIN,), jnp.int32),
        pltpu.VMEM_SHARED((_NPAD,), jnp.float32),
        pltpu.SemaphoreType.DMA,
        pltpu.SemaphoreType.DMA,
    ],
)
def _deg_kernel(eif, deg_out, zbuf, ones, idxf, deg_sp, semst, sem):
    cid = lax.axis_index("c")
    sid = lax.axis_index("s")
    gid = cid * 16 + sid
    nw = _WBASE + jnp.where(gid < _WEXTRA, 1, 0)
    base_e = (gid * _WBASE + jnp.minimum(gid, _WEXTRA)) * _WIN

    _stage(eif, _E, base_e, gid, idxf, semst)
    _zero_vmem(zbuf, _CHUNK)
    pltpu.sync_copy(zbuf, deg_sp.at[pl.ds(sid * _CHUNK, _CHUNK)])
    for i in range(_WIN // 16):
        ones[pl.ds(i * 16, 16)] = jnp.ones((16,), jnp.float32)
    _stage_wait(eif, _E, base_e, gid, idxf, semst)
    plsc.subcore_barrier()

    def fire(j, c):
        pltpu.async_copy(ones, deg_sp.at[idxf.at[pl.ds(j * _WIN, _WIN)]],
                         sem, add=True)
        return c

    lax.fori_loop(0, nw, fire, 0)

    def drain(j, c):
        pltpu.make_async_copy(ones,
                              deg_sp.at[idxf.at[pl.ds(j * _WIN, _WIN)]],
                              sem).wait()
        return c

    lax.fori_loop(0, nw, drain, 0)

    plsc.subcore_barrier()
    pltpu.sync_copy(deg_sp.at[pl.ds(sid * _CHUNK, _CHUNK)], zbuf)
    pltpu.sync_copy(zbuf, deg_out.at[cid, pl.ds(sid * _CHUNK, _CHUNK)])


@functools.partial(
    pl.kernel,
    mesh=_mesh,
    out_type=jax.ShapeDtypeStruct((2, _NPAD), jnp.float32),
    scratch_types=[
        pltpu.VMEM((_CHUNK,), jnp.float32),
        pltpu.VMEM((_CHUNK,), jnp.float32),
        pltpu.VMEM((_CHUNK,), jnp.float32),
        pltpu.VMEM((_EBASE + _WIN,), jnp.float32),
        pltpu.VMEM((_EBASE + _WIN,), jnp.int32),
        pltpu.VMEM((_EBASE + _WIN,), jnp.int32),
        pltpu.VMEM_SHARED((_NPAD,), jnp.float32),
        pltpu.VMEM_SHARED((_NPAD,), jnp.float32),
        pltpu.SemaphoreType.DMA,
        pltpu.SemaphoreType.DMA,
        pltpu.SemaphoreType.DMA,
        pltpu.SemaphoreType.DMA,
    ],
)
def _coef_kernel(eif, deg_part, coef_out,
                 zb, d0, d1, wtf, idxdf, idxsf, invd_sp, coef_sp,
                 semst, semtb, semg, sems):
    cid = lax.axis_index("c")
    sid = lax.axis_index("s")
    gid = cid * 16 + sid
    nw = _WBASE + jnp.where(gid < _WEXTRA, 1, 0)
    base_e = (gid * _WBASE + jnp.minimum(gid, _WEXTRA)) * _WIN

    _stage(eif, _E, base_e, gid, idxdf, semst)
    _stage(eif, 0, base_e, gid, idxsf, semst)
    pltpu.async_copy(deg_part.at[0, pl.ds(sid * _CHUNK, _CHUNK)], d0, semtb)
    pltpu.async_copy(deg_part.at[1, pl.ds(sid * _CHUNK, _CHUNK)], d1, semtb)

    _zero_vmem(zb, _CHUNK)
    pltpu.sync_copy(zb, coef_sp.at[pl.ds(sid * _CHUNK, _CHUNK)])

    pltpu.make_async_copy(deg_part.at[0, pl.ds(sid * _CHUNK, _CHUNK)],
                          d0, semtb).wait()
    pltpu.make_async_copy(deg_part.at[1, pl.ds(sid * _CHUNK, _CHUNK)],
                          d1, semtb).wait()
    one = jnp.ones((16,), jnp.float32)
    for i in range(_CHUNK // 16):
        s = pl.ds(i * 16, 16)
        d0[s] = one / jnp.maximum(d0[s] + d1[s], one)
    pltpu.sync_copy(d0, invd_sp.at[pl.ds(sid * _CHUNK, _CHUNK)])

    _stage_wait(eif, _E, base_e, gid, idxdf, semst)
    _stage_wait(eif, 0, base_e, gid, idxsf, semst)
    plsc.subcore_barrier()

    def fire_g(j, c):
        pltpu.async_copy(invd_sp.at[idxdf.at[pl.ds(j * _WIN, _WIN)]],
                         wtf.at[pl.ds(j * _WIN, _WIN)], semg)
        return c

    lax.fori_loop(0, nw, fire_g, 0)

    def chain(j, c):
        pltpu.make_async_copy(invd_sp.at[idxdf.at[pl.ds(j * _WIN, _WIN)]],
                              wtf.at[pl.ds(j * _WIN, _WIN)], semg).wait()
        pltpu.async_copy(wtf.at[pl.ds(j * _WIN, _WIN)],
                         coef_sp.at[idxsf.at[pl.ds(j * _WIN, _WIN)]],
                         sems, add=True)
        return c

    lax.fori_loop(0, nw, chain, 0)

    def drain(j, c):
        pltpu.make_async_copy(wtf.at[pl.ds(j * _WIN, _WIN)],
                              coef_sp.at[idxsf.at[pl.ds(j * _WIN, _WIN)]],
                              sems).wait()
        return c

    lax.fori_loop(0, nw, drain, 0)

    plsc.subcore_barrier()
    pltpu.sync_copy(coef_sp.at[pl.ds(sid * _CHUNK, _CHUNK)], zb)
    pltpu.sync_copy(zb, coef_out.at[cid, pl.ds(sid * _CHUNK, _CHUNK)])


def _tc_body(cp_ref, x_ref, w_ref, b_ref, o_ref):
    coef = cp_ref[0:1, :_N] + cp_ref[1:2, :_N]
    v = jnp.dot(coef, x_ref[...], preferred_element_type=jnp.float32)
    csum = jnp.sum(coef)
    o_ref[...] = (jnp.dot(v, w_ref[...], preferred_element_type=jnp.float32)
                  + csum * b_ref[...]) * (1.0 / _N)


def kernel(x, edge_index, W, b):
    eif = edge_index.reshape(2 * _E)
    deg_part = _deg_kernel(eif)
    coef_part = _coef_kernel(eif, deg_part)
    out = pl.pallas_call(
        _tc_body,
        out_shape=jax.ShapeDtypeStruct((1, _C), jnp.float32),
    )(coef_part, x, W, b.reshape(1, _C))
    return out
